# Initial kernel scaffold; baseline (speedup 1.0000x reference)
import functools

import jax
import jax.numpy as jnp
from jax import lax
from jax.experimental import pallas as pl
from jax.experimental.pallas import tpu as pltpu
from jax.experimental.pallas import tpu_sc as plsc

VOCAB = 1000000
EMB = 32
BATCH = 4096
SEQ = 200
N = BATCH * SEQ
NW = 32
PER_W = N // NW
CHUNK = 128
G = PER_W // CHUNK
K = 8
L = 16


def _make_kernel():
    mesh = plsc.VectorSubcoreMesh(core_axis_name="c", subcore_axis_name="s")

    @functools.partial(
        pl.kernel,
        out_type=(
            jax.ShapeDtypeStruct((NW, G, CHUNK, EMB), jnp.float32),
            jax.ShapeDtypeStruct((NW, G, CHUNK), jnp.float32),
        ),
        mesh=mesh,
        scratch_types=(
            [
                pltpu.VMEM((G, CHUNK), jnp.int32),
                pltpu.VMEM((K, CHUNK, EMB), jnp.float32),
                pltpu.VMEM((G, CHUNK), jnp.float32),
            ]
            + [pltpu.SemaphoreType.DMA] * K
            + [pltpu.SemaphoreType.DMA]
        ),
    )
    def emb_kernel(x_hbm, w_hbm, out_hbm, mask_hbm, idx_v, rows_v, mask_v,
                   *sems):
        gsems = sems[:K]
        wid = lax.axis_index("s") * 2 + lax.axis_index("c")

        pltpu.sync_copy(x_hbm.at[wid], idx_v)

        for r in range(K):
            pltpu.async_copy(w_hbm.at[idx_v.at[r]], rows_v.at[r], gsems[r])

        def step(s, _):
            for r in range(K):
                g = s * K + r
                for j in range(CHUNK // L):
                    v = idx_v[g, pl.ds(j * L, L)]
                    mask_v[g, pl.ds(j * L, L)] = jnp.where(
                        v != 0, jnp.float32(1.0), jnp.float32(0.0))
                pltpu.make_async_copy(
                    w_hbm.at[idx_v.at[g]], rows_v.at[r], gsems[r]).wait()
                pltpu.sync_copy(rows_v.at[r], out_hbm.at[wid, g])
                nxt = g + K

                @pl.when(nxt < G)
                def _():
                    pltpu.async_copy(
                        w_hbm.at[idx_v.at[nxt]], rows_v.at[r], gsems[r])

            return 0

        lax.fori_loop(0, G // K, step, 0)
        pltpu.sync_copy(mask_v, mask_hbm.at[wid])

    return emb_kernel


_emb_kernel = None


def kernel(x, W):
    global _emb_kernel
    if _emb_kernel is None:
        _emb_kernel = _make_kernel()
    xf = x.reshape(NW, G, CHUNK).astype(jnp.int32)
    emb, mask = _emb_kernel(xf, W)
    return emb.reshape(BATCH, SEQ, EMB), mask.reshape(BATCH, SEQ)


# speedup vs baseline: 1.4994x; 1.4994x over previous
import functools

import jax
import jax.numpy as jnp
from jax import lax
from jax.experimental import pallas as pl
from jax.experimental.pallas import tpu as pltpu
from jax.experimental.pallas import tpu_sc as plsc

VOCAB = 1000000
EMB = 32
BATCH = 4096
SEQ = 200
N = BATCH * SEQ
NW = 32
PER_W = N // NW
CHUNK = 128
G = PER_W // CHUNK
K = 8
L = 16


def _make_kernel():
    mesh = plsc.VectorSubcoreMesh(core_axis_name="c", subcore_axis_name="s")

    @functools.partial(
        pl.kernel,
        out_type=(
            jax.ShapeDtypeStruct((NW, G, CHUNK, EMB), jnp.float32),
            jax.ShapeDtypeStruct((NW, G, CHUNK), jnp.float32),
        ),
        mesh=mesh,
        compiler_params=pltpu.CompilerParams(use_tc_tiling_on_sc=False),
        scratch_types=(
            [
                pltpu.VMEM((G, CHUNK), jnp.int32),
                pltpu.VMEM((K, CHUNK, EMB), jnp.float32),
                pltpu.VMEM((G, CHUNK), jnp.float32),
            ]
            + [pltpu.SemaphoreType.DMA] * K
            + [pltpu.SemaphoreType.DMA]
        ),
    )
    def emb_kernel(x_hbm, w_hbm, out_hbm, mask_hbm, idx_v, rows_v, mask_v,
                   *sems):
        gsems = sems[:K]
        wid = lax.axis_index("s") * 2 + lax.axis_index("c")

        pltpu.sync_copy(x_hbm.at[wid], idx_v)

        for r in range(K):
            pltpu.async_copy(w_hbm.at[idx_v.at[r]], rows_v.at[r], gsems[r])

        def step(s, _):
            for r in range(K):
                g = s * K + r
                for j in range(CHUNK // L):
                    v = idx_v[g, pl.ds(j * L, L)]
                    mask_v[g, pl.ds(j * L, L)] = jnp.where(
                        v != 0, jnp.float32(1.0), jnp.float32(0.0))
                pltpu.make_async_copy(
                    w_hbm.at[idx_v.at[g]], rows_v.at[r], gsems[r]).wait()
                pltpu.sync_copy(rows_v.at[r], out_hbm.at[wid, g])
                nxt = g + K

                @pl.when(nxt < G)
                def _():
                    pltpu.async_copy(
                        w_hbm.at[idx_v.at[nxt]], rows_v.at[r], gsems[r])

            return 0

        lax.fori_loop(0, G // K, step, 0)
        pltpu.sync_copy(mask_v, mask_hbm.at[wid])

    return emb_kernel


_emb_kernel = None


def kernel(x, W):
    global _emb_kernel
    if _emb_kernel is None:
        _emb_kernel = _make_kernel()
    xf = x.reshape(NW, G, CHUNK).astype(jnp.int32)
    emb, mask = _emb_kernel(xf, W)
    return emb.reshape(BATCH, SEQ, EMB), mask.reshape(BATCH, SEQ)
